# Initial kernel scaffold; baseline (speedup 1.0000x reference)
import functools

import jax
import jax.numpy as jnp
from jax import lax
from jax.experimental import pallas as pl
from jax.experimental.pallas import tpu as pltpu
from jax.experimental.pallas import tpu_sc as plsc

N = 10000
NPAD = 10240
D = 128
E = 320000
NC = 2
NS = 16
NW = NC * NS
EPW = E // NW
CHUNK = 80
NCH = EPW // CHUNK
NBUF = 5
RPT = NPAD // NS
RBLK = 1024

_mesh = plsc.VectorSubcoreMesh(
    core_axis_name="c", subcore_axis_name="s", num_cores=NC, num_subcores=NS)


@functools.partial(
    pl.kernel,
    out_type=jax.ShapeDtypeStruct((NC, NPAD), jnp.float32),
    mesh=_mesh,
    scratch_types=[
        pltpu.VMEM((NCH, CHUNK), jnp.int32),
        pltpu.VMEM((CHUNK,), jnp.float32),
        pltpu.VMEM((RPT,), jnp.float32),
        pltpu.VMEM_SHARED((NPAD,), jnp.float32),
        pltpu.SemaphoreType.DMA,
    ],
)
def _deg_call(dst_hbm, out_hbm, dst_v, ones_v, zero_v, acc, dsem):
    c = lax.axis_index("c")
    s = lax.axis_index("s")
    wid = s * NC + c
    pltpu.sync_copy(dst_hbm.at[wid], dst_v)
    for i in range(CHUNK // 16):
        ones_v[pl.ds(i * 16, 16)] = jnp.full((16,), 1.0, jnp.float32)
    for i in range(RPT // 16):
        zero_v[pl.ds(i * 16, 16)] = jnp.zeros((16,), jnp.float32)
    pltpu.sync_copy(zero_v, acc.at[pl.ds(s * RPT, RPT)])
    plsc.subcore_barrier()

    K = 8

    def _start(j):
        pltpu.async_copy(ones_v, acc.at[dst_v.at[j]], dsem, add=True)

    def _wait(j):
        pltpu.make_async_copy(ones_v, acc.at[dst_v.at[j]], dsem).wait()

    def body(j, _):
        _start(j)

        @pl.when(j >= K)
        def _():
            _wait(j - K)
        return _
    lax.fori_loop(0, NCH, body, None)
    for i in range(K):
        _wait(NCH - K + i)
    plsc.subcore_barrier()
    pltpu.sync_copy(acc.at[pl.ds(s * RPT, RPT)],
                    out_hbm.at[c, pl.ds(s * RPT, RPT)])


@functools.partial(
    pl.kernel,
    out_type=jax.ShapeDtypeStruct((NC, NPAD, D), jnp.float32),
    mesh=_mesh,
    scratch_types=[
        pltpu.VMEM((EPW,), jnp.int32),
        pltpu.VMEM((NCH, CHUNK), jnp.int32),
        pltpu.VMEM((NBUF, CHUNK, D), jnp.float32),
        pltpu.VMEM_SHARED((NPAD, D), jnp.float32),
        pltpu.SemaphoreType.DMA((NBUF,)),
        pltpu.SemaphoreType.DMA((NBUF,)),
    ],
)
def _agg_call(y_hbm, src_hbm, dst_hbm, zrows_hbm, out_hbm,
              src_v, dst_v, rows_v, acc, gsem, ssem):
    c = lax.axis_index("c")
    s = lax.axis_index("s")
    wid = s * NC + c
    pltpu.sync_copy(src_hbm.at[wid], src_v)
    pltpu.sync_copy(dst_hbm.at[wid], dst_v)
    pltpu.sync_copy(zrows_hbm, acc.at[pl.ds(s * RPT, RPT)])
    plsc.subcore_barrier()

    def gather(j, b):
        return pltpu.make_async_copy(
            y_hbm.at[src_v.at[pl.ds(j * CHUNK, CHUNK)]],
            rows_v.at[b], gsem.at[b])

    for b in range(NBUF):
        gather(b, b).start()

    def outer(g, _):
        for b in range(NBUF):
            j = g * NBUF + b
            gather(j, b).wait()
            pltpu.async_copy(rows_v.at[b], acc.at[dst_v.at[j]],
                             ssem.at[b], add=True)
            pltpu.make_async_copy(rows_v.at[b], acc.at[dst_v.at[j]],
                                  ssem.at[b]).wait()

            @pl.when(j + NBUF < NCH)
            def _():
                gather(j + NBUF, b).start()
        return _
    lax.fori_loop(0, NCH // NBUF, outer, None)
    plsc.subcore_barrier()
    pltpu.sync_copy(acc.at[pl.ds(s * RPT, RPT)],
                    out_hbm.at[c, pl.ds(s * RPT, RPT)])


def _scale_mm_body(x_ref, w_ref, degp_ref, y_ref):
    dis = lax.rsqrt(degp_ref[0, :] + degp_ref[1, :] + 1.0)
    xw = jnp.dot(x_ref[...], w_ref[...], preferred_element_type=jnp.float32)
    y_ref[...] = xw * dis[:, None]


def _layer2_body(p0_ref, p1_ref, y_ref, degp_ref, b_ref, w_ref, o_ref):
    dis = lax.rsqrt(degp_ref[0, :] + degp_ref[1, :] + 1.0)
    h = dis[:, None] * (p0_ref[...] + p1_ref[...] + y_ref[...]) + b_ref[...]
    h = jnp.maximum(h, 0.0)
    o_ref[...] = dis[:, None] * jnp.dot(
        h, w_ref[...], preferred_element_type=jnp.float32)


def _combine_body(p0_ref, p1_ref, y_ref, degp_ref, b_ref, o_ref):
    dis = lax.rsqrt(degp_ref[0, :] + degp_ref[1, :] + 1.0)
    o_ref[...] = dis[:, None] * (p0_ref[...] + p1_ref[...] + y_ref[...]) \
        + b_ref[...]


_row_spec = pl.BlockSpec((RBLK, D), lambda i: (i, 0))
_mat_spec = pl.BlockSpec((D, D), lambda i: (0, 0))
_deg_spec = pl.BlockSpec((NC, RBLK), lambda i: (0, i))
_bias_spec = pl.BlockSpec((1, D), lambda i: (0, 0))
_rows_out = jax.ShapeDtypeStruct((NPAD, D), jnp.float32)
_GRID = NPAD // RBLK


def _scale_mm(xpad, W, degp):
    return pl.pallas_call(
        _scale_mm_body, grid=(_GRID,),
        in_specs=[_row_spec, _mat_spec, _deg_spec],
        out_specs=_row_spec, out_shape=_rows_out)(xpad, W, degp)


def _layer2(p, y1, degp, b1, W2):
    return pl.pallas_call(
        _layer2_body, grid=(_GRID,),
        in_specs=[_row_spec, _row_spec, _row_spec, _deg_spec,
                  _bias_spec, _mat_spec],
        out_specs=_row_spec, out_shape=_rows_out)(
            p[0], p[1], y1, degp, b1.reshape(1, D), W2)


def _combine(p, y2, degp, b2):
    return pl.pallas_call(
        _combine_body, grid=(_GRID,),
        in_specs=[_row_spec, _row_spec, _row_spec, _deg_spec, _bias_spec],
        out_specs=_row_spec, out_shape=_rows_out)(
            p[0], p[1], y2, degp, b2.reshape(1, D))


def kernel(x, edge_index, W1, b1, W2, b2):
    ei = edge_index.astype(jnp.int32)
    src = ei[0].reshape(NW, EPW)
    dst = ei[1].reshape(NW, NCH, CHUNK)
    xpad = jnp.zeros((NPAD, D), jnp.float32).at[:N].set(x)
    zrows = jnp.zeros((RPT, D), jnp.float32)

    degp = _deg_call(dst)
    y1 = _scale_mm(xpad, W1, degp)
    p1 = _agg_call(y1, src, dst, zrows)
    y2 = _layer2(p1, y1, degp, b1, W2)
    p2 = _agg_call(y2, src, dst, zrows)
    out = _combine(p2, y2, degp, b2)
    return out[:N]


# speedup vs baseline: 36.0312x; 36.0312x over previous
import functools

import jax
import jax.numpy as jnp
from jax import lax
from jax.experimental import pallas as pl
from jax.experimental.pallas import tpu as pltpu
from jax.experimental.pallas import tpu_sc as plsc

N = 10000
NPAD = 10240
D = 128
E = 320000
NC = 2
NS = 16
NW = NC * NS
EPW = E // NW
CHUNK = 80
NCH = EPW // CHUNK
CHA = 40
NCHA = EPW // CHA
NBUF = 5
RPT = NPAD // NS
RBLK = 1024

_mesh = plsc.VectorSubcoreMesh(
    core_axis_name="c", subcore_axis_name="s", num_cores=NC, num_subcores=NS)


@functools.partial(
    pl.kernel,
    out_type=jax.ShapeDtypeStruct((NC, NPAD), jnp.float32),
    mesh=_mesh,
    scratch_types=[
        pltpu.VMEM((NCH, CHUNK), jnp.int32),
        pltpu.VMEM((CHUNK,), jnp.float32),
        pltpu.VMEM((RPT,), jnp.float32),
        pltpu.VMEM_SHARED((NPAD,), jnp.float32),
        pltpu.SemaphoreType.DMA,
    ],
)
def _deg_call(dst_hbm, out_hbm, dst_v, ones_v, zero_v, acc, dsem):
    c = lax.axis_index("c")
    s = lax.axis_index("s")
    wid = s * NC + c
    pltpu.sync_copy(dst_hbm.at[wid], dst_v)
    for i in range(CHUNK // 16):
        ones_v[pl.ds(i * 16, 16)] = jnp.full((16,), 1.0, jnp.float32)
    for i in range(RPT // 16):
        zero_v[pl.ds(i * 16, 16)] = jnp.zeros((16,), jnp.float32)
    pltpu.sync_copy(zero_v, acc.at[pl.ds(s * RPT, RPT)])
    plsc.subcore_barrier()

    K = 8

    def _start(j):
        pltpu.async_copy(ones_v, acc.at[dst_v.at[j]], dsem, add=True)

    def _wait(j):
        pltpu.make_async_copy(ones_v, acc.at[dst_v.at[j]], dsem).wait()

    def body(j, _):
        _start(j)

        @pl.when(j >= K)
        def _():
            _wait(j - K)
        return _
    lax.fori_loop(0, NCH, body, None)
    for i in range(K):
        _wait(NCH - K + i)
    plsc.subcore_barrier()
    pltpu.sync_copy(acc.at[pl.ds(s * RPT, RPT)],
                    out_hbm.at[c, pl.ds(s * RPT, RPT)])


@functools.partial(
    pl.kernel,
    out_type=jax.ShapeDtypeStruct((NC, NPAD, D), jnp.float32),
    mesh=_mesh,
    scratch_types=[
        pltpu.VMEM((EPW,), jnp.int32),
        pltpu.VMEM((NBUF, CHA), jnp.int32),
        pltpu.VMEM((NBUF, CHA, D), jnp.float32),
        pltpu.VMEM_SHARED((NPAD, D), jnp.float32),
        pltpu.SemaphoreType.DMA((NBUF,)),
        pltpu.SemaphoreType.DMA((NBUF,)),
        pltpu.SemaphoreType.DMA((NBUF,)),
    ],
)
def _agg_call(y_hbm, src_hbm, dst_hbm, zrows_hbm, out_hbm,
              src_v, dst_v, rows_v, acc, gsem, ssem, isem):
    c = lax.axis_index("c")
    s = lax.axis_index("s")
    wid = s * NC + c
    pltpu.sync_copy(src_hbm.at[wid], src_v)
    pltpu.sync_copy(zrows_hbm, acc.at[pl.ds(s * RPT, RPT)])
    plsc.subcore_barrier()

    def gather(j, b):
        return pltpu.make_async_copy(
            y_hbm.at[src_v.at[pl.ds(j * CHA, CHA)]],
            rows_v.at[b], gsem.at[b])

    def idx_copy(j, b):
        return pltpu.make_async_copy(
            dst_hbm.at[wid, j], dst_v.at[b], isem.at[b])

    for b in range(NBUF):
        idx_copy(b, b).start()
        gather(b, b).start()

    def outer(g, _):
        for b in range(NBUF):
            j = g * NBUF + b
            idx_copy(j, b).wait()
            gather(j, b).wait()
            pltpu.async_copy(rows_v.at[b], acc.at[dst_v.at[b]],
                             ssem.at[b], add=True)
            pltpu.make_async_copy(rows_v.at[b], acc.at[dst_v.at[b]],
                                  ssem.at[b]).wait()

            @pl.when(j + NBUF < NCHA)
            def _():
                idx_copy(j + NBUF, b).start()
                gather(j + NBUF, b).start()
        return _
    lax.fori_loop(0, NCHA // NBUF, outer, None)
    plsc.subcore_barrier()
    pltpu.sync_copy(acc.at[pl.ds(s * RPT, RPT)],
                    out_hbm.at[c, pl.ds(s * RPT, RPT)])


def _scale_mm_body(x_ref, w_ref, degp_ref, y_ref):
    dis = lax.rsqrt(degp_ref[0, :] + degp_ref[1, :] + 1.0)
    xw = jnp.dot(x_ref[...], w_ref[...], preferred_element_type=jnp.float32)
    y_ref[...] = xw * dis[:, None]


def _layer2_body(p0_ref, p1_ref, y_ref, degp_ref, b_ref, w_ref, o_ref):
    dis = lax.rsqrt(degp_ref[0, :] + degp_ref[1, :] + 1.0)
    h = dis[:, None] * (p0_ref[...] + p1_ref[...] + y_ref[...]) + b_ref[...]
    h = jnp.maximum(h, 0.0)
    o_ref[...] = dis[:, None] * jnp.dot(
        h, w_ref[...], preferred_element_type=jnp.float32)


def _combine_body(p0_ref, p1_ref, y_ref, degp_ref, b_ref, o_ref):
    dis = lax.rsqrt(degp_ref[0, :] + degp_ref[1, :] + 1.0)
    o_ref[...] = dis[:, None] * (p0_ref[...] + p1_ref[...] + y_ref[...]) \
        + b_ref[...]


_row_spec = pl.BlockSpec((RBLK, D), lambda i: (i, 0))
_mat_spec = pl.BlockSpec((D, D), lambda i: (0, 0))
_deg_spec = pl.BlockSpec((NC, RBLK), lambda i: (0, i))
_bias_spec = pl.BlockSpec((1, D), lambda i: (0, 0))
_rows_out = jax.ShapeDtypeStruct((NPAD, D), jnp.float32)
_GRID = NPAD // RBLK


def _scale_mm(xpad, W, degp):
    return pl.pallas_call(
        _scale_mm_body, grid=(_GRID,),
        in_specs=[_row_spec, _mat_spec, _deg_spec],
        out_specs=_row_spec, out_shape=_rows_out)(xpad, W, degp)


def _layer2(p, y1, degp, b1, W2):
    return pl.pallas_call(
        _layer2_body, grid=(_GRID,),
        in_specs=[_row_spec, _row_spec, _row_spec, _deg_spec,
                  _bias_spec, _mat_spec],
        out_specs=_row_spec, out_shape=_rows_out)(
            p[0], p[1], y1, degp, b1.reshape(1, D), W2)


def _combine(p, y2, degp, b2):
    return pl.pallas_call(
        _combine_body, grid=(_GRID,),
        in_specs=[_row_spec, _row_spec, _row_spec, _deg_spec, _bias_spec],
        out_specs=_row_spec, out_shape=_rows_out)(
            p[0], p[1], y2, degp, b2.reshape(1, D))


def kernel(x, edge_index, W1, b1, W2, b2):
    ei = edge_index.astype(jnp.int32)
    src = ei[0].reshape(NW, EPW)
    dst_deg = ei[1].reshape(NW, NCH, CHUNK)
    dst_agg = ei[1].reshape(NW, NCHA, CHA)
    xpad = jnp.zeros((NPAD, D), jnp.float32).at[:N].set(x)
    zrows = jnp.zeros((RPT, D), jnp.float32)

    degp = _deg_call(dst_deg)
    y1 = _scale_mm(xpad, W1, degp)
    p1 = _agg_call(y1, src, dst_agg, zrows)
    y2 = _layer2(p1, y1, degp, b1, W2)
    p2 = _agg_call(y2, src, dst_agg, zrows)
    out = _combine(p2, y2, degp, b2)
    return out[:N]
